# baseline (speedup 1.0000x reference)
import functools

import jax
import jax.numpy as jnp
from jax import lax
from jax.experimental import pallas as pl
from jax.experimental.pallas import tpu as pltpu
from jax.experimental.pallas import tpu_sc as plsc

NC = 2
NS = 16
NW = NC * NS
CHUNK = 128
NBUF = 8


@functools.partial(jax.jit, static_argnames=("total", "embed_dim", "j_per_w", "num_fields"))
def _fused_gather(cat3, weight, *, total, embed_dim, j_per_w, num_fields):
    b_per_w = j_per_w * CHUNK
    per_table = weight.shape[0] // num_fields
    mesh = plsc.VectorSubcoreMesh(core_axis_name="c", subcore_axis_name="s")
    n_rounds = j_per_w // NBUF

    @functools.partial(
        pl.kernel,
        out_type=jax.ShapeDtypeStruct((total, embed_dim), jnp.float32),
        mesh=mesh,
        compiler_params=pltpu.CompilerParams(use_tc_tiling_on_sc=False),
        scratch_types=[
            pltpu.VMEM((j_per_w, CHUNK), jnp.int32),
            pltpu.VMEM((NBUF, CHUNK, embed_dim), jnp.float32),
            pltpu.SemaphoreType.DMA((NBUF,)),
            pltpu.SemaphoreType.DMA((NBUF,)),
        ],
    )
    def run(cat_hbm, w_hbm, out_hbm, idx_v, rows_v, gsem, wsem):
        wid = lax.axis_index("s") * NC + lax.axis_index("c")
        pltpu.sync_copy(cat_hbm.at[wid], idx_v)

        lane = lax.iota(jnp.int32, 16)
        steps_per_row = CHUNK // 16

        def add_body(t, f):
            j = t // steps_per_row
            i = (t % steps_per_row) * 16
            idx_v[j, pl.ds(i, 16)] = idx_v[j, pl.ds(i, 16)] + f * per_table
            fn = f + 16
            return jnp.where(fn >= num_fields, fn - num_fields, fn)

        lax.fori_loop(0, j_per_w * steps_per_row, add_body,
                      lane % num_fields)

        base = wid * b_per_w

        def gather_args(j, b):
            return (w_hbm.at[idx_v.at[j]], rows_v.at[b], gsem.at[b])

        def write_args(j, b):
            return (rows_v.at[b],
                    out_hbm.at[pl.ds(base + j * CHUNK, CHUNK)], wsem.at[b])

        for b in range(NBUF):
            pltpu.async_copy(*gather_args(b, b))

        def round_body(g, refill):
            j0 = g * NBUF
            for b in range(NBUF):
                pltpu.make_async_copy(*gather_args(j0 + b, b)).wait()
                pltpu.async_copy(*write_args(j0 + b, b))
            for b in range(NBUF):
                pltpu.make_async_copy(*write_args(j0 + b, b)).wait()
                if refill:
                    pltpu.async_copy(*gather_args(j0 + NBUF + b, b))

        pl.loop(0, n_rounds - 1)(lambda g: round_body(g, True))
        round_body(n_rounds - 1, False)

    return run(cat3, weight)


def kernel(categorical_inputs, weight):
    B, F = categorical_inputs.shape
    R, D = weight.shape
    total = B * F
    assert total % (NW * CHUNK) == 0
    j_per_w = total // (NW * CHUNK)
    assert j_per_w % NBUF == 0

    cat3 = categorical_inputs.reshape(NW, j_per_w, CHUNK)
    out = _fused_gather(cat3, weight, total=total, embed_dim=D,
                        j_per_w=j_per_w, num_fields=F)
    return out.reshape(B, F, D)
